# baseline (speedup 1.0000x reference)
import jax
import jax.numpy as jnp
from jax.experimental import pallas as pl
from jax.experimental.pallas import tpu as pltpu

R, C = 128, 100000
NCH = 16
BR = R // NCH

def _copy(x_ref, o_ref, *sems):
    copies = []
    for k in range(NCH):
        cp = pltpu.make_async_copy(
            x_ref.at[pl.ds(k * BR, BR)],
            o_ref.at[pl.ds(k * BR, BR)],
            sems[k],
        )
        cp.start()
        copies.append(cp)
    for cp in copies:
        cp.wait()

@jax.jit
def kernel(inputs):
    return pl.pallas_call(
        _copy,
        in_specs=[pl.BlockSpec(memory_space=pltpu.MemorySpace.HBM)],
        out_specs=pl.BlockSpec(memory_space=pltpu.MemorySpace.HBM),
        out_shape=jax.ShapeDtypeStruct((R, C), jnp.float32),
        scratch_shapes=[pltpu.SemaphoreType.DMA] * NCH,
    )(inputs)


# speedup vs baseline: 12.8746x; 12.8746x over previous
import jax
import jax.numpy as jnp
from jax.experimental import pallas as pl
from jax.experimental.pallas import tpu as pltpu

R, C = 128, 100000
BR = 16
GRID = R // BR


def _softmax_block(x_ref, o_ref):
    x = x_ref[...]
    m = jnp.max(x, axis=1, keepdims=True)
    e = jnp.exp(x - m)
    s = jnp.sum(e, axis=1, keepdims=True)
    o_ref[...] = e * (1.0 / s)


@jax.jit
def kernel(inputs):
    return pl.pallas_call(
        _softmax_block,
        grid=(GRID,),
        in_specs=[pl.BlockSpec((BR, C), lambda i: (i, 0))],
        out_specs=pl.BlockSpec((BR, C), lambda i: (i, 0)),
        out_shape=jax.ShapeDtypeStruct((R, C), jnp.float32),
        compiler_params=pltpu.CompilerParams(
            dimension_semantics=("arbitrary",),
        ),
    )(inputs)
